# Initial kernel scaffold; baseline (speedup 1.0000x reference)
import jax
import jax.numpy as jnp
from jax.experimental import pallas as pl


def kernel(x, edge_index, batch, target, W1, a_src1, a_dst1, b1, W2, a_src2, a_dst2, b2, fcg1_W, fcg1_b, emb, attn_Win, attn_Wout, conv_W, conv_b, fcxt_W, fcxt_b, fc1_W, fc1_b, fc2_W, fc2_b, out_W, out_b):
    raise NotImplementedError("write your pallas kernel here")


# speedup vs baseline: 5.8431x; 5.8431x over previous
import jax
import jax.numpy as jnp
from jax.experimental import pallas as pl
from jax.experimental.pallas import tpu as pltpu

_CHUNK = 512


def _attn_kernel(off_ref, mc_ref, h2_ref, q_ref, out_ref):
    b = pl.program_id(0)
    start = off_ref[b]
    n = off_ref[b + 1] - start
    qb = q_ref[0]
    s_len = qb.shape[0]
    m0 = jnp.full((1, s_len), -1e30, jnp.float32)
    d0 = jnp.zeros((1, s_len), jnp.float32)
    a0 = jnp.zeros((s_len, qb.shape[1]), jnp.float32)
    nchunks = (n + _CHUNK - 1) // _CHUNK

    def body(i, carry):
        m, d, acc = carry
        rs = start + i * _CHUNK
        hc = h2_ref[pl.ds(rs, _CHUNK), :]
        s = jax.lax.dot_general(hc, qb, (((1,), (1,)), ((), ())),
                                preferred_element_type=jnp.float32)
        e = i * _CHUNK + jax.lax.broadcasted_iota(jnp.int32, (_CHUNK, 1), 0)
        s = jnp.where(e < n, s, -1e30)
        cm = jnp.max(s, axis=0, keepdims=True)
        mn = jnp.maximum(m, cm)
        corr = jnp.exp(m - mn)
        p = jnp.exp(s - mn)
        dn = d * corr + jnp.sum(p, axis=0, keepdims=True)
        accn = acc * corr.T + jax.lax.dot_general(
            p, hc, (((0,), (0,)), ((), ())), preferred_element_type=jnp.float32)
        return mn, dn, accn

    m, d, acc = jax.lax.fori_loop(0, nchunks, body, (m0, d0, a0))
    mf = jnp.maximum(m, 0.0)
    scale = jnp.exp(m - mf)
    pad = (mc_ref[0] - n).astype(jnp.float32)
    den = d * scale + pad * jnp.exp(-mf)
    out_ref[0] = acc * (scale / den).T


def _attention(h2p, q, offsets, maxc):
    B, S, D = q.shape
    return pl.pallas_call(
        _attn_kernel,
        grid=(B,),
        in_specs=[
            pl.BlockSpec(memory_space=pltpu.SMEM),
            pl.BlockSpec(memory_space=pltpu.SMEM),
            pl.BlockSpec(h2p.shape, lambda b: (0, 0)),
            pl.BlockSpec((1, S, D), lambda b: (b, 0, 0)),
        ],
        out_specs=pl.BlockSpec((1, S, D), lambda b: (b, 0, 0)),
        out_shape=jax.ShapeDtypeStruct((B, S, D), jnp.float32),
    )(offsets, maxc, h2p, q)


def _gat(x, src, dst, N, W, a_src, a_dst, bias, heads, out_dim):
    h = (x @ W).reshape(N, heads, out_dim)
    as_ = jnp.sum(h * a_src[None, :, :], axis=-1)
    ad_ = jnp.sum(h * a_dst[None, :, :], axis=-1)
    e = as_[src] + ad_[dst]
    e = jnp.where(e > 0, e, 0.2 * e)
    emax = jax.ops.segment_max(e, dst, num_segments=N)
    emax = jnp.where(jnp.isfinite(emax), emax, 0.0)
    ex = jnp.exp(e - emax[dst])
    den = jax.ops.segment_sum(ex, dst, num_segments=N)
    alpha = ex / (den[dst] + 1e-16)
    out = jax.ops.segment_sum(h[src] * alpha[:, :, None], dst, num_segments=N)
    return out.reshape(N, heads * out_dim) + bias


def kernel(x, edge_index, batch, target, W1, a_src1, a_dst1, b1, W2, a_src2, a_dst2, b2, fcg1_W, fcg1_b, emb, attn_Win, attn_Wout, conv_W, conv_b, fcxt_W, fcxt_b, fc1_W, fc1_b, fc2_W, fc2_b, out_W, out_b):
    N = x.shape[0]
    B, S = target.shape
    loop = jnp.arange(N)
    src = jnp.concatenate([edge_index[0], loop])
    dst = jnp.concatenate([edge_index[1], loop])
    h1 = jax.nn.elu(_gat(x, src, dst, N, W1, a_src1, a_dst1, b1, 10, 78))
    h2 = jax.nn.relu(_gat(h1, src, dst, N, W2, a_src2, a_dst2, b2, 1, 128))

    counts = jnp.bincount(batch, length=B)
    offsets = jnp.concatenate([jnp.zeros((1,), jnp.int32),
                               jnp.cumsum(counts).astype(jnp.int32)])
    maxc = jnp.max(counts).astype(jnp.int32).reshape(1)

    emb_xt = emb[target]
    q = emb_xt @ attn_Win

    padn = ((N + _CHUNK - 1) // _CHUNK + 1) * _CHUNK
    h2p = jnp.zeros((padn, h2.shape[1]), h2.dtype).at[:N].set(h2)
    mix = _attention(h2p, q, offsets, maxc)

    attn_out = jnp.tanh(jnp.concatenate([mix, emb_xt], axis=-1) @ attn_Wout)

    xg = jax.ops.segment_max(h2, batch, num_segments=B)
    xg = jnp.where(jnp.isfinite(xg), xg, 0.0)
    xg = jax.nn.relu(xg @ fcg1_W + fcg1_b)

    conv = jax.lax.conv_general_dilated(attn_out, conv_W, (1,), 'VALID',
                                        dimension_numbers=('NCH', 'OIH', 'NCH'))
    conv = jax.nn.relu(conv + conv_b[None, :, None])
    xt = conv.reshape(B, -1) @ fcxt_W + fcxt_b
    xc = jnp.concatenate([xg, xt], axis=1)
    xc = jax.nn.relu(xc @ fc1_W + fc1_b)
    xc = jax.nn.relu(xc @ fc2_W + fc2_b)
    return xc @ out_W + out_b


# speedup vs baseline: 21.7142x; 3.7162x over previous
import jax
import jax.numpy as jnp
from jax.experimental import pallas as pl
from jax.experimental.pallas import tpu as pltpu

_CHUNK = 512


def _attn_kernel(off_ref, mc_ref, h2_ref, q_ref, out_ref):
    b = pl.program_id(0)
    start = off_ref[b]
    n = off_ref[b + 1] - start
    qb = q_ref[0]
    s_len = qb.shape[0]
    m0 = jnp.full((1, s_len), -1e30, jnp.float32)
    d0 = jnp.zeros((1, s_len), jnp.float32)
    a0 = jnp.zeros((s_len, qb.shape[1]), jnp.float32)
    nchunks = (n + _CHUNK - 1) // _CHUNK

    def body(i, carry):
        m, d, acc = carry
        rs = start + i * _CHUNK
        hc = h2_ref[pl.ds(rs, _CHUNK), :]
        s = jax.lax.dot_general(hc, qb, (((1,), (1,)), ((), ())),
                                preferred_element_type=jnp.float32)
        e = i * _CHUNK + jax.lax.broadcasted_iota(jnp.int32, (_CHUNK, 1), 0)
        s = jnp.where(e < n, s, -1e30)
        cm = jnp.max(s, axis=0, keepdims=True)
        mn = jnp.maximum(m, cm)
        corr = jnp.exp(m - mn)
        p = jnp.exp(s - mn)
        dn = d * corr + jnp.sum(p, axis=0, keepdims=True)
        accn = acc * corr.T + jax.lax.dot_general(
            p, hc, (((0,), (0,)), ((), ())), preferred_element_type=jnp.float32)
        return mn, dn, accn

    m, d, acc = jax.lax.fori_loop(0, nchunks, body, (m0, d0, a0))
    mf = jnp.maximum(m, 0.0)
    scale = jnp.exp(m - mf)
    pad = (mc_ref[0] - n).astype(jnp.float32)
    den = d * scale + pad * jnp.exp(-mf)
    out_ref[0] = acc * (scale / den).T


def _attention(h2p, q, offsets, maxc):
    B, S, D = q.shape
    return pl.pallas_call(
        _attn_kernel,
        grid=(B,),
        in_specs=[
            pl.BlockSpec(memory_space=pltpu.SMEM),
            pl.BlockSpec(memory_space=pltpu.SMEM),
            pl.BlockSpec(h2p.shape, lambda b: (0, 0)),
            pl.BlockSpec((1, S, D), lambda b: (b, 0, 0)),
        ],
        out_specs=pl.BlockSpec((1, S, D), lambda b: (b, 0, 0)),
        out_shape=jax.ShapeDtypeStruct((B, S, D), jnp.float32),
    )(offsets, maxc, h2p, q)


_ECHUNK = 512
_NBLK = 256


def _segsum_kernel(heads, fh, rpb_ref, hrows_hbm, ea_hbm, sd_hbm, out_ref,
                   hbuf, ebuf, dbuf, sem_h, sem_e, sem_d):
    i = pl.program_id(0)
    d0 = i * _NBLK
    e_lo = rpb_ref[i]
    e_hi = rpb_ref[i + 1]
    start = (e_lo // _ECHUNK) * _ECHUNK
    nch = (e_hi - start + _ECHUNK - 1) // _ECHUNK
    acc0 = tuple(jnp.zeros((_NBLK, fh), jnp.float32) for _ in range(heads))
    riota = jax.lax.broadcasted_iota(jnp.int32, (_ECHUNK, _NBLK), 1)

    def body(j, accs):
        off = start + j * _ECHUNK
        ch = pltpu.make_async_copy(hrows_hbm.at[pl.ds(off, _ECHUNK), :], hbuf, sem_h)
        ce = pltpu.make_async_copy(ea_hbm.at[pl.ds(off, _ECHUNK), :], ebuf, sem_e)
        cd = pltpu.make_async_copy(sd_hbm.at[pl.ds(off, _ECHUNK), :], dbuf, sem_d)
        ch.start(); ce.start(); cd.start()
        ch.wait(); ce.wait(); cd.wait()
        ld = dbuf[...] - d0
        valid = (ld >= 0) & (ld < _NBLK)
        ea = ebuf[...]
        ve = jnp.exp(jnp.where(ea > 0, ea, 0.2 * ea))
        ve = jnp.where(valid, ve, 0.0)
        oh = ld == riota
        hv = hbuf[...]
        new = []
        for h in range(heads):
            w = jnp.where(oh, ve[:, h:h + 1], 0.0)
            new.append(accs[h] + jax.lax.dot_general(
                w, hv[:, h * fh:(h + 1) * fh], (((0,), (0,)), ((), ())),
                preferred_element_type=jnp.float32))
        return tuple(new)

    accs = jax.lax.fori_loop(0, nch, body, acc0)
    out_ref[...] = jnp.concatenate(accs, axis=1) if heads > 1 else accs[0]


def _segsum(hrows, ea, sd2, rpb, heads, fh, nblocks):
    import functools
    f = heads * fh
    kern = functools.partial(_segsum_kernel, heads, fh)
    return pl.pallas_call(
        kern,
        grid=(nblocks,),
        in_specs=[
            pl.BlockSpec(memory_space=pltpu.SMEM),
            pl.BlockSpec(memory_space=pl.ANY),
            pl.BlockSpec(memory_space=pl.ANY),
            pl.BlockSpec(memory_space=pl.ANY),
        ],
        out_specs=pl.BlockSpec((_NBLK, f), lambda i: (i, 0)),
        out_shape=jax.ShapeDtypeStruct((nblocks * _NBLK, f), jnp.float32),
        scratch_shapes=[
            pltpu.VMEM((_ECHUNK, f), jnp.float32),
            pltpu.VMEM((_ECHUNK, 16), jnp.float32),
            pltpu.VMEM((_ECHUNK, 1), jnp.int32),
            pltpu.SemaphoreType.DMA,
            pltpu.SemaphoreType.DMA,
            pltpu.SemaphoreType.DMA,
        ],
    )(rpb, hrows, ea, sd2)


def _gat(x, ssrc, sdst, sd2, rpb, N, W, a_src, a_dst, bias, heads, out_dim):
    fh = ((out_dim + 1 + 15) // 16) * 16
    h = (x @ W).reshape(N, heads, out_dim)
    as_ = jnp.sum(h * a_src[None, :, :], axis=-1)
    ad_ = jnp.sum(h * a_dst[None, :, :], axis=-1)
    hp = jnp.zeros((N, heads, fh), jnp.float32)
    hp = hp.at[:, :, :out_dim].set(h).at[:, :, out_dim].set(1.0)
    hrows = hp.reshape(N, heads * fh)[ssrc]
    ea = as_[ssrc] + ad_[jnp.minimum(sdst, N - 1)]
    eap = jnp.zeros((ssrc.shape[0], 16), jnp.float32).at[:, :heads].set(ea)
    nblocks = rpb.shape[0] - 1
    seg = _segsum(hrows, eap, sd2, rpb, heads, fh, nblocks)
    seg = seg[:N].reshape(N, heads, fh)
    out = seg[:, :, :out_dim] / seg[:, :, out_dim:out_dim + 1]
    return out.reshape(N, heads * out_dim) + bias


def kernel(x, edge_index, batch, target, W1, a_src1, a_dst1, b1, W2, a_src2, a_dst2, b2, fcg1_W, fcg1_b, emb, attn_Win, attn_Wout, conv_W, conv_b, fcxt_W, fcxt_b, fc1_W, fc1_b, fc2_W, fc2_b, out_W, out_b):
    N = x.shape[0]
    B, S = target.shape
    loop = jnp.arange(N)
    src = jnp.concatenate([edge_index[0], loop])
    dst = jnp.concatenate([edge_index[1], loop])

    E2 = src.shape[0]
    order = jnp.argsort(dst)
    sdst = dst[order].astype(jnp.int32)
    ssrc = src[order].astype(jnp.int32)
    e2p = ((E2 + _ECHUNK - 1) // _ECHUNK + 1) * _ECHUNK
    sdst_p = jnp.full((e2p,), jnp.int32(2 ** 30)).at[:E2].set(sdst)
    ssrc_p = jnp.zeros((e2p,), jnp.int32).at[:E2].set(ssrc)
    sd2 = sdst_p.reshape(e2p, 1)
    nblocks = (N + _NBLK - 1) // _NBLK
    rpb = jnp.searchsorted(sdst, jnp.arange(nblocks + 1, dtype=jnp.int32) * _NBLK
                           ).astype(jnp.int32)

    h1 = jax.nn.elu(_gat(x, ssrc_p, sdst_p, sd2, rpb, N, W1, a_src1, a_dst1, b1, 10, 78))
    h2 = jax.nn.relu(_gat(h1, ssrc_p, sdst_p, sd2, rpb, N, W2, a_src2, a_dst2, b2, 1, 128))

    counts = jnp.bincount(batch, length=B)
    offsets = jnp.concatenate([jnp.zeros((1,), jnp.int32),
                               jnp.cumsum(counts).astype(jnp.int32)])
    maxc = jnp.max(counts).astype(jnp.int32).reshape(1)

    emb_xt = emb[target]
    q = emb_xt @ attn_Win

    padn = ((N + _CHUNK - 1) // _CHUNK + 1) * _CHUNK
    h2p = jnp.zeros((padn, h2.shape[1]), h2.dtype).at[:N].set(h2)
    mix = _attention(h2p, q, offsets, maxc)

    attn_out = jnp.tanh(jnp.concatenate([mix, emb_xt], axis=-1) @ attn_Wout)

    xg = jax.ops.segment_max(h2, batch, num_segments=B)
    xg = jnp.where(jnp.isfinite(xg), xg, 0.0)
    xg = jax.nn.relu(xg @ fcg1_W + fcg1_b)

    conv = jax.lax.conv_general_dilated(attn_out, conv_W, (1,), 'VALID',
                                        dimension_numbers=('NCH', 'OIH', 'NCH'))
    conv = jax.nn.relu(conv + conv_b[None, :, None])
    xt = conv.reshape(B, -1) @ fcxt_W + fcxt_b
    xc = jnp.concatenate([xg, xt], axis=1)
    xc = jax.nn.relu(xc @ fc1_W + fc1_b)
    xc = jax.nn.relu(xc @ fc2_W + fc2_b)
    return xc @ out_W + out_b


# speedup vs baseline: 22.5386x; 1.0380x over previous
import jax
import jax.numpy as jnp
from jax.experimental import pallas as pl
from jax.experimental.pallas import tpu as pltpu

_CHUNK = 512


def _attn_kernel(off_ref, mc_ref, h2_ref, q_ref, out_ref):
    b = pl.program_id(0)
    start = off_ref[b]
    n = off_ref[b + 1] - start
    qb = q_ref[0]
    s_len = qb.shape[0]
    m0 = jnp.full((1, s_len), -1e30, jnp.float32)
    d0 = jnp.zeros((1, s_len), jnp.float32)
    a0 = jnp.zeros((s_len, qb.shape[1]), jnp.float32)
    nchunks = (n + _CHUNK - 1) // _CHUNK

    def body(i, carry):
        m, d, acc = carry
        rs = start + i * _CHUNK
        hc = h2_ref[pl.ds(rs, _CHUNK), :]
        s = jax.lax.dot_general(hc, qb, (((1,), (1,)), ((), ())),
                                preferred_element_type=jnp.float32)
        e = i * _CHUNK + jax.lax.broadcasted_iota(jnp.int32, (_CHUNK, 1), 0)
        s = jnp.where(e < n, s, -1e30)
        cm = jnp.max(s, axis=0, keepdims=True)
        mn = jnp.maximum(m, cm)
        corr = jnp.exp(m - mn)
        p = jnp.exp(s - mn)
        dn = d * corr + jnp.sum(p, axis=0, keepdims=True)
        accn = acc * corr.T + jax.lax.dot_general(
            p, hc, (((0,), (0,)), ((), ())), preferred_element_type=jnp.float32)
        return mn, dn, accn

    m, d, acc = jax.lax.fori_loop(0, nchunks, body, (m0, d0, a0))
    mf = jnp.maximum(m, 0.0)
    scale = jnp.exp(m - mf)
    pad = (mc_ref[0] - n).astype(jnp.float32)
    den = d * scale + pad * jnp.exp(-mf)
    out_ref[0] = acc * (scale / den).T


def _attention(h2p, q, offsets, maxc):
    B, S, D = q.shape
    return pl.pallas_call(
        _attn_kernel,
        grid=(B,),
        in_specs=[
            pl.BlockSpec(memory_space=pltpu.SMEM),
            pl.BlockSpec(memory_space=pltpu.SMEM),
            pl.BlockSpec(h2p.shape, lambda b: (0, 0)),
            pl.BlockSpec((1, S, D), lambda b: (b, 0, 0)),
        ],
        out_specs=pl.BlockSpec((1, S, D), lambda b: (b, 0, 0)),
        out_shape=jax.ShapeDtypeStruct((B, S, D), jnp.float32),
        compiler_params=pltpu.CompilerParams(
            dimension_semantics=("parallel",)),
    )(offsets, maxc, h2p, q)


_ECHUNK = 1024
_NBLK = 256


def _segsum_kernel(heads, fh, rpb_ref, hrows_hbm, ea_hbm, sd_hbm, out_ref,
                   hbuf, ebuf, dbuf, sem_h, sem_e, sem_d):
    i = pl.program_id(0)
    d0 = i * _NBLK
    e_lo = rpb_ref[i]
    e_hi = rpb_ref[i + 1]
    start = (e_lo // _ECHUNK) * _ECHUNK
    nch = (e_hi - start + _ECHUNK - 1) // _ECHUNK
    acc0 = tuple(jnp.zeros((_NBLK, fh), jnp.float32) for _ in range(heads))
    riota = jax.lax.broadcasted_iota(jnp.int32, (_ECHUNK, _NBLK), 1)

    def body(j, accs):
        off = start + j * _ECHUNK
        ch = pltpu.make_async_copy(hrows_hbm.at[pl.ds(off, _ECHUNK), :], hbuf, sem_h)
        ce = pltpu.make_async_copy(ea_hbm.at[pl.ds(off, _ECHUNK), :], ebuf, sem_e)
        cd = pltpu.make_async_copy(sd_hbm.at[pl.ds(off, _ECHUNK), :], dbuf, sem_d)
        ch.start(); ce.start(); cd.start()
        ch.wait(); ce.wait(); cd.wait()
        ld = dbuf[...] - d0
        valid = (ld >= 0) & (ld < _NBLK)
        ea = ebuf[...]
        ve = jnp.exp(jnp.where(ea > 0, ea, 0.2 * ea))
        ve = jnp.where(valid, ve, 0.0)
        oh = ld == riota
        hv = hbuf[...]
        new = []
        for h in range(heads):
            w = jnp.where(oh, ve[:, h:h + 1], 0.0)
            new.append(accs[h] + jax.lax.dot_general(
                w, hv[:, h * fh:(h + 1) * fh], (((0,), (0,)), ((), ())),
                preferred_element_type=jnp.float32))
        return tuple(new)

    accs = jax.lax.fori_loop(0, nch, body, acc0)
    out_ref[...] = jnp.concatenate(accs, axis=1) if heads > 1 else accs[0]


def _segsum(hrows, ea, sd2, rpb, heads, fh, nblocks):
    import functools
    f = heads * fh
    kern = functools.partial(_segsum_kernel, heads, fh)
    return pl.pallas_call(
        kern,
        grid=(nblocks,),
        in_specs=[
            pl.BlockSpec(memory_space=pltpu.SMEM),
            pl.BlockSpec(memory_space=pl.ANY),
            pl.BlockSpec(memory_space=pl.ANY),
            pl.BlockSpec(memory_space=pl.ANY),
        ],
        out_specs=pl.BlockSpec((_NBLK, f), lambda i: (i, 0)),
        out_shape=jax.ShapeDtypeStruct((nblocks * _NBLK, f), jnp.float32),
        scratch_shapes=[
            pltpu.VMEM((_ECHUNK, f), jnp.float32),
            pltpu.VMEM((_ECHUNK, 16), jnp.float32),
            pltpu.VMEM((_ECHUNK, 1), jnp.int32),
            pltpu.SemaphoreType.DMA,
            pltpu.SemaphoreType.DMA,
            pltpu.SemaphoreType.DMA,
        ],
        compiler_params=pltpu.CompilerParams(
            dimension_semantics=("parallel",)),
    )(rpb, hrows, ea, sd2)


def _gat(x, ssrc, sdst, sd2, rpb, N, W, a_src, a_dst, bias, heads, out_dim):
    fh = ((out_dim + 1 + 15) // 16) * 16
    h = (x @ W).reshape(N, heads, out_dim)
    as_ = jnp.sum(h * a_src[None, :, :], axis=-1)
    ad_ = jnp.sum(h * a_dst[None, :, :], axis=-1)
    hp = jnp.zeros((N, heads, fh), jnp.float32)
    hp = hp.at[:, :, :out_dim].set(h).at[:, :, out_dim].set(1.0)
    hrows = hp.reshape(N, heads * fh)[ssrc]
    ea = as_[ssrc] + ad_[jnp.minimum(sdst, N - 1)]
    eap = jnp.zeros((ssrc.shape[0], 16), jnp.float32).at[:, :heads].set(ea)
    nblocks = rpb.shape[0] - 1
    seg = _segsum(hrows, eap, sd2, rpb, heads, fh, nblocks)
    seg = seg[:N].reshape(N, heads, fh)
    out = seg[:, :, :out_dim] / seg[:, :, out_dim:out_dim + 1]
    return out.reshape(N, heads * out_dim) + bias


def kernel(x, edge_index, batch, target, W1, a_src1, a_dst1, b1, W2, a_src2, a_dst2, b2, fcg1_W, fcg1_b, emb, attn_Win, attn_Wout, conv_W, conv_b, fcxt_W, fcxt_b, fc1_W, fc1_b, fc2_W, fc2_b, out_W, out_b):
    N = x.shape[0]
    B, S = target.shape
    loop = jnp.arange(N)
    src = jnp.concatenate([edge_index[0], loop])
    dst = jnp.concatenate([edge_index[1], loop])

    E2 = src.shape[0]
    order = jnp.argsort(dst)
    sdst = dst[order].astype(jnp.int32)
    ssrc = src[order].astype(jnp.int32)
    e2p = ((E2 + _ECHUNK - 1) // _ECHUNK + 1) * _ECHUNK
    sdst_p = jnp.full((e2p,), jnp.int32(2 ** 30)).at[:E2].set(sdst)
    ssrc_p = jnp.zeros((e2p,), jnp.int32).at[:E2].set(ssrc)
    sd2 = sdst_p.reshape(e2p, 1)
    nblocks = (N + _NBLK - 1) // _NBLK
    rpb = jnp.searchsorted(sdst, jnp.arange(nblocks + 1, dtype=jnp.int32) * _NBLK
                           ).astype(jnp.int32)

    h1 = jax.nn.elu(_gat(x, ssrc_p, sdst_p, sd2, rpb, N, W1, a_src1, a_dst1, b1, 10, 78))
    h2 = jax.nn.relu(_gat(h1, ssrc_p, sdst_p, sd2, rpb, N, W2, a_src2, a_dst2, b2, 1, 128))

    counts = jnp.bincount(batch, length=B)
    offsets = jnp.concatenate([jnp.zeros((1,), jnp.int32),
                               jnp.cumsum(counts).astype(jnp.int32)])
    maxc = jnp.max(counts).astype(jnp.int32).reshape(1)

    emb_xt = emb[target]
    q = emb_xt @ attn_Win

    padn = ((N + _CHUNK - 1) // _CHUNK + 1) * _CHUNK
    h2p = jnp.zeros((padn, h2.shape[1]), h2.dtype).at[:N].set(h2)
    mix = _attention(h2p, q, offsets, maxc)

    attn_out = jnp.tanh(jnp.concatenate([mix, emb_xt], axis=-1) @ attn_Wout)

    xg = jax.ops.segment_max(h2, batch, num_segments=B)
    xg = jnp.where(jnp.isfinite(xg), xg, 0.0)
    xg = jax.nn.relu(xg @ fcg1_W + fcg1_b)

    conv = jax.lax.conv_general_dilated(attn_out, conv_W, (1,), 'VALID',
                                        dimension_numbers=('NCH', 'OIH', 'NCH'))
    conv = jax.nn.relu(conv + conv_b[None, :, None])
    xt = conv.reshape(B, -1) @ fcxt_W + fcxt_b
    xc = jnp.concatenate([xg, xt], axis=1)
    xc = jax.nn.relu(xc @ fc1_W + fc1_b)
    xc = jax.nn.relu(xc @ fc2_W + fc2_b)
    return xc @ out_W + out_b
